# baseline (speedup 1.0000x reference)
import functools

import jax
import jax.numpy as jnp
from jax import lax
from jax.experimental import pallas as pl
from jax.experimental.pallas import tpu as pltpu
from jax.experimental.pallas import tpu_sc as plsc

_R = 64
_K = 32
_L = 16
_NC = 2
_NS = 16
_NW = _NC * _NS


def _mm_body(x_ref, wt_ref, out_ref):
    out_ref[...] = jnp.dot(x_ref[...], wt_ref[...],
                           preferred_element_type=jnp.float32)


def _matmul(x2, wt, block_rows):
    n, d = x2.shape
    return pl.pallas_call(
        _mm_body,
        grid=(n // block_rows,),
        in_specs=[
            pl.BlockSpec((block_rows, d), lambda i: (i, 0)),
            pl.BlockSpec((d, _R), lambda i: (0, 0)),
        ],
        out_specs=pl.BlockSpec((block_rows, _R), lambda i: (i, 0)),
        out_shape=jax.ShapeDtypeStruct((n, _R), jnp.float32),
    )(x2, wt)


def _merge16(ka, va, kb, vb):
    rkb = lax.rev(kb, (0,))
    rvb = lax.rev(vb, (0,))
    m = ka >= rkb
    hk = jnp.where(m, ka, rkb)
    hv = jnp.where(m, va, rvb)
    lk = jnp.where(m, rkb, ka)
    lv = jnp.where(m, rvb, va)
    hk, hv = plsc.sort_key_val(hk, hv, descending=True)
    lk, lv = plsc.sort_key_val(lk, lv, descending=True)
    return hk, hv, lk, lv


def _topk_row(k0, k1, k2, k3, iota):
    qs = []
    for q, kq in enumerate((k0, k1, k2, k3)):
        sk, sv = plsc.sort_key_val(kq, iota + (q * _L), descending=True)
        qs.append((sk, sv))
    ahk, ahv, alk, alv = _merge16(qs[0][0], qs[0][1], qs[1][0], qs[1][1])
    bhk, bhv, blk, blv = _merge16(qs[2][0], qs[2][1], qs[3][0], qs[3][1])
    rb0 = lax.rev(blk, (0,))
    rv0 = lax.rev(blv, (0,))
    rb1 = lax.rev(bhk, (0,))
    rv1 = lax.rev(bhv, (0,))
    m0 = ahk >= rb0
    h0k = jnp.where(m0, ahk, rb0)
    h0v = jnp.where(m0, ahv, rv0)
    m1 = alk >= rb1
    h1k = jnp.where(m1, alk, rb1)
    h1v = jnp.where(m1, alv, rv1)
    mm = h0k >= h1k
    hhk = jnp.where(mm, h0k, h1k)
    hhv = jnp.where(mm, h0v, h1v)
    hlk = jnp.where(mm, h1k, h0k)
    hlv = jnp.where(mm, h1v, h0v)
    hhk, hhv = plsc.sort_key_val(hhk, hhv, descending=True)
    hlk, hlv = plsc.sort_key_val(hlk, hlv, descending=True)
    return hhk, hhv, hlk, hlv


def _sc_topk(acts_flat, n_rows):
    rows_w = n_rows // _NW
    mesh = plsc.VectorSubcoreMesh(core_axis_name="c", subcore_axis_name="s",
                                  num_cores=_NC, num_subcores=_NS)

    @functools.partial(
        pl.kernel,
        out_type=(
            jax.ShapeDtypeStruct((n_rows * _K,), jnp.int32),
            jax.ShapeDtypeStruct((n_rows * _K,), jnp.float32),
        ),
        mesh=mesh,
        scratch_types=[
            pltpu.VMEM((rows_w * _R,), jnp.float32),
            pltpu.VMEM((rows_w * _K,), jnp.int32),
            pltpu.VMEM((rows_w * _K,), jnp.float32),
        ],
        compiler_params=pltpu.CompilerParams(needs_layout_passes=False),
    )
    def k(acts_hbm, idx_hbm, val_hbm, acts_v, idx_v, val_v):
        wid = lax.axis_index("s") * _NC + lax.axis_index("c")
        pltpu.sync_copy(acts_hbm.at[pl.ds(wid * rows_w * _R, rows_w * _R)],
                        acts_v)
        iota = lax.broadcasted_iota(jnp.int32, (_L,), 0)

        def body(r, carry):
            base = r * _R
            k0 = acts_v[pl.ds(base, _L)]
            k1 = acts_v[pl.ds(base + _L, _L)]
            k2 = acts_v[pl.ds(base + 2 * _L, _L)]
            k3 = acts_v[pl.ds(base + 3 * _L, _L)]
            hhk, hhv, hlk, hlv = _topk_row(k0, k1, k2, k3, iota)
            ob = r * _K
            val_v[pl.ds(ob, _L)] = hhk
            val_v[pl.ds(ob + _L, _L)] = hlk
            idx_v[pl.ds(ob, _L)] = hhv
            idx_v[pl.ds(ob + _L, _L)] = hlv
            return carry

        lax.fori_loop(0, rows_w, body, 0)
        pltpu.sync_copy(idx_v, idx_hbm.at[pl.ds(wid * rows_w * _K,
                                                rows_w * _K)])
        pltpu.sync_copy(val_v, val_hbm.at[pl.ds(wid * rows_w * _K,
                                                rows_w * _K)])

    return k(acts_flat)


@jax.jit
def kernel(x, W_A):
    b, s, d = x.shape
    n = b * s
    x2 = x.reshape(n, d)
    acts = _matmul(x2, W_A.T, block_rows=512)
    idx_flat, val_flat = _sc_topk(acts.reshape(-1), n)
    return (idx_flat.reshape(n, _K),
            val_flat.reshape(n, _K),
            acts.reshape(b, s, _R))
